# Initial kernel scaffold; baseline (speedup 1.0000x reference)
import jax
import jax.numpy as jnp
from jax.experimental import pallas as pl


def kernel(src, table, gamma, beta):
    raise NotImplementedError("write your pallas kernel here")


# speedup vs baseline: 1.9083x; 1.9083x over previous
import functools

import jax
import jax.numpy as jnp
from jax import lax
from jax.experimental import pallas as pl
from jax.experimental.pallas import tpu as pltpu
from jax.experimental.pallas import tpu_sc as plsc

EPS = 1e-6
L = 16
CH = 128
NW = 32


def _rsqrt_newton(xv):
    i = lax.bitcast_convert_type(xv, jnp.int32)
    i = jnp.int32(0x5F3759DF) - lax.shift_right_arithmetic(i, jnp.int32(1))
    y = lax.bitcast_convert_type(i, jnp.float32)
    half = xv * 0.5
    for _ in range(3):
        y = y * (1.5 - half * y * y)
    return y


_GATHER_DNUMS = lax.GatherDimensionNumbers(
    offset_dims=(), collapsed_slice_dims=(0,), start_index_map=(0,))


def _dyn_gather(x, idx):
    return lax.gather(
        x, idx[:, None], _GATHER_DNUMS, slice_sizes=(1,),
        mode=lax.GatherScatterMode.PROMISE_IN_BOUNDS)


def _xlane_sum(x, perms):
    for p in perms:
        x = x + _dyn_gather(x, p)
    return x


def _make_sc_kernel(n_rows, dim):
    rows_per_w = n_rows // NW
    chunks = rows_per_w // CH
    nk = dim // L
    mesh = plsc.VectorSubcoreMesh(core_axis_name="c", subcore_axis_name="s")

    @functools.partial(
        pl.kernel,
        out_type=jax.ShapeDtypeStruct((n_rows, dim), jnp.float32),
        mesh=mesh,
        scratch_types=[
            pltpu.VMEM((rows_per_w,), jnp.int32),
            pltpu.VMEM((CH, dim), jnp.float32),
            pltpu.VMEM((dim,), jnp.float32),
            pltpu.VMEM((dim,), jnp.float32),
            pltpu.SemaphoreType.DMA,
        ],
    )
    def kern(table_hbm, idx_hbm, gamma_hbm, beta_hbm, out_hbm,
             idx_v, buf, gamma_v, beta_v, sem):
        wid = lax.axis_index("s") * 2 + lax.axis_index("c")
        pltpu.sync_copy(idx_hbm.at[pl.ds(wid * rows_per_w, rows_per_w)],
                        idx_v)
        pltpu.sync_copy(gamma_hbm, gamma_v)
        pltpu.sync_copy(beta_hbm, beta_v)
        gv = [gamma_v[pl.ds(k * L, L)] for k in range(nk)]
        bv = [beta_v[pl.ds(k * L, L)] for k in range(nk)]
        inv_d = jnp.float32(1.0 / dim)
        row0 = wid * rows_per_w
        lane = lax.iota(jnp.int32, L)
        perms = [lane ^ sh for sh in (8, 4, 2, 1)]

        def chunk_body(j, _):
            pltpu.async_copy(table_hbm.at[idx_v.at[pl.ds(j * CH, CH)]],
                             buf, sem).wait()

            def row_body(r, _):
                xs = [buf[r, pl.ds(k * L, L)] for k in range(nk)]
                s = xs[0]
                ss = xs[0] * xs[0]
                for k in range(1, nk):
                    s = s + xs[k]
                    ss = ss + xs[k] * xs[k]
                tot = _xlane_sum(s, perms)
                tot2 = _xlane_sum(ss, perms)
                mean_v = tot * inv_d
                var_v = tot2 * inv_d - mean_v * mean_v
                rstd_v = _rsqrt_newton(var_v + EPS)
                for k in range(nk):
                    buf[r, pl.ds(k * L, L)] = (
                        (xs[k] - mean_v) * rstd_v * gv[k] + bv[k])
                return 0

            lax.fori_loop(0, CH, row_body, 0)
            pltpu.sync_copy(buf, out_hbm.at[pl.ds(row0 + j * CH, CH)])
            return 0

        lax.fori_loop(0, chunks, chunk_body, 0)

    return kern


def kernel(src, table, gamma, beta):
    b, s = src.shape
    v, d = table.shape
    idx = src.reshape(-1).astype(jnp.int32)
    n = idx.shape[0]
    out = _make_sc_kernel(n, d)(table, idx, gamma, beta)
    return out.reshape(b, s, d)


# speedup vs baseline: 3.4903x; 1.8290x over previous
import functools

import jax
import jax.numpy as jnp
from jax import lax
from jax.experimental import pallas as pl
from jax.experimental.pallas import tpu as pltpu
from jax.experimental.pallas import tpu_sc as plsc

EPS = 1e-6
L = 16
CH = 128
NW = 32

_GATHER_DNUMS = lax.GatherDimensionNumbers(
    offset_dims=(), collapsed_slice_dims=(0,), start_index_map=(0,))


def _dyn_gather(x, idx):
    return lax.gather(
        x, idx[:, None], _GATHER_DNUMS, slice_sizes=(1,),
        mode=lax.GatherScatterMode.PROMISE_IN_BOUNDS)


def _xlane_sum(x, perms):
    for p in perms:
        x = x + _dyn_gather(x, p)
    return x


def _rsqrt_newton(xv):
    i = lax.bitcast_convert_type(xv, jnp.int32)
    i = jnp.int32(0x5F3759DF) - lax.shift_right_arithmetic(i, jnp.int32(1))
    y = lax.bitcast_convert_type(i, jnp.float32)
    half = xv * 0.5
    for _ in range(2):
        y = y * (1.5 - half * y * y)
    return y


def _make_sc_kernel(n_rows, dim):
    rows_per_w = n_rows // NW
    chunks = rows_per_w // CH
    nk = dim // L
    mesh = plsc.VectorSubcoreMesh(core_axis_name="c", subcore_axis_name="s")

    @functools.partial(
        pl.kernel,
        out_type=jax.ShapeDtypeStruct((n_rows, dim), jnp.float32),
        mesh=mesh,
        scratch_types=[
            pltpu.VMEM((rows_per_w,), jnp.int32),
            pltpu.VMEM((CH, dim), jnp.float32),
            pltpu.VMEM((CH, dim), jnp.float32),
            pltpu.VMEM((CH, dim), jnp.float32),
            pltpu.VMEM((CH, dim), jnp.float32),
            pltpu.VMEM((dim,), jnp.float32),
            pltpu.VMEM((dim,), jnp.float32),
            pltpu.SemaphoreType.DMA,
            pltpu.SemaphoreType.DMA,
            pltpu.SemaphoreType.DMA,
            pltpu.SemaphoreType.DMA,
        ],
    )
    def kern(table_hbm, idx_hbm, gamma_hbm, beta_hbm, out_hbm,
             idx_v, gb0, gb1, ob0, ob1, gamma_v, beta_v,
             gs0, gs1, os0, os1):
        wid = lax.axis_index("s") * 2 + lax.axis_index("c")
        pltpu.sync_copy(idx_hbm.at[pl.ds(wid * rows_per_w, rows_per_w)],
                        idx_v)
        pltpu.sync_copy(gamma_hbm, gamma_v)
        pltpu.sync_copy(beta_hbm, beta_v)
        gv = [gamma_v[pl.ds(k * L, L)] for k in range(nk)]
        bv = [beta_v[pl.ds(k * L, L)] for k in range(nk)]
        inv_d = jnp.float32(1.0 / dim)
        row0 = wid * rows_per_w
        lane = lax.iota(jnp.int32, L)
        perms = [lane ^ sh for sh in (8, 4, 2, 1)]

        gbuf = (gb0, gb1)
        obuf = (ob0, ob1)
        gsem = (gs0, gs1)
        osem = (os0, os1)

        def issue_gather(c, b):
            pltpu.async_copy(table_hbm.at[idx_v.at[pl.ds(c * CH, CH)]],
                             gbuf[b], gsem[b])

        def wait_gather(b):
            pltpu.make_async_copy(table_hbm.at[pl.ds(0, CH)], gbuf[b],
                                  gsem[b]).wait()

        def issue_out(c, b):
            pltpu.async_copy(obuf[b],
                             out_hbm.at[pl.ds(row0 + c * CH, CH)], osem[b])

        def wait_out(b):
            pltpu.make_async_copy(obuf[b], out_hbm.at[pl.ds(row0, CH)],
                                  osem[b]).wait()

        def ln_row(gb, ob, r):
            xs = [gb[r, pl.ds(k * L, L)] for k in range(nk)]
            s01, s23 = xs[0] + xs[1], xs[2] + xs[3]
            s45, s67 = xs[4] + xs[5], xs[6] + xs[7]
            s = (s01 + s23) + (s45 + s67)
            sq = [x * x for x in xs]
            q01, q23 = sq[0] + sq[1], sq[2] + sq[3]
            q45, q67 = sq[4] + sq[5], sq[6] + sq[7]
            ss = (q01 + q23) + (q45 + q67)
            mean = _xlane_sum(s, perms) * inv_d
            var = _xlane_sum(ss, perms) * inv_d - mean * mean
            rstd = _rsqrt_newton(var + EPS)
            for k in range(nk):
                ob[r, pl.ds(k * L, L)] = (
                    (xs[k] - mean) * rstd * gv[k] + bv[k])

        def compute(b):
            gb, ob = gbuf[b], obuf[b]

            def row_body(i, _):
                r = i * 2
                ln_row(gb, ob, r)
                ln_row(gb, ob, r + 1)
                return 0

            lax.fori_loop(0, CH // 2, row_body, 0)

        issue_gather(0, 0)
        issue_gather(1, 1)
        for c in (0, 1):
            b = c & 1
            wait_gather(b)
            compute(b)
            issue_gather(c + 2, b)
            issue_out(c, b)

        def pair_body(p, _):
            for b in (0, 1):
                c = 2 + p * 2 + b
                wait_gather(b)
                compute(b)
                issue_gather(c + 2, b)
                wait_out(b)
                issue_out(c, b)
            return 0

        lax.fori_loop(0, (chunks - 4) // 2, pair_body, 0)

        for c in (chunks - 2, chunks - 1):
            b = c & 1
            wait_gather(b)
            compute(b)
            wait_out(b)
            issue_out(c, b)
        wait_out(0)
        wait_out(1)

    return kern


def kernel(src, table, gamma, beta):
    b, s = src.shape
    v, d = table.shape
    idx = src.reshape(-1).astype(jnp.int32)
    n = idx.shape[0]
    out = _make_sc_kernel(n, d)(table, idx, gamma, beta)
    return out.reshape(b, s, d)
